# Initial kernel scaffold; baseline (speedup 1.0000x reference)
import jax
import jax.numpy as jnp
from jax.experimental import pallas as pl


def kernel(xs, h0, whx, bhx, whh, bhh, woh, boh):
    raise NotImplementedError("write your pallas kernel here")


# speedup vs baseline: 2.2632x; 2.2632x over previous
import functools

import jax
import jax.numpy as jnp
from jax.experimental import pallas as pl
from jax.experimental.pallas import tpu as pltpu


_K = 16


def _f32dot(a, b):
    return jnp.dot(a, b, preferred_element_type=jnp.float32)


def _prep_kernel(whx_ref, bhx_ref, whh_ref, bhh_ref, woh_ref,
                 wbf_ref, wxp_ref, bp_ref, wk_ref, z_ref, wohbf_ref):
    w = whh_ref[...].astype(jnp.bfloat16)
    wbf_ref[...] = w
    wxp_ref[...] = _f32dot(whx_ref[...].astype(jnp.bfloat16), w).astype(jnp.bfloat16)
    bp_ref[...] = _f32dot(bhx_ref[...].astype(jnp.bfloat16), w) + bhh_ref[...]
    woh_bf = woh_ref[...].astype(jnp.bfloat16)
    wohbf_ref[...] = woh_bf

    w2 = _f32dot(w, w).astype(jnp.bfloat16)
    w4 = _f32dot(w2, w2).astype(jnp.bfloat16)
    w8 = _f32dot(w4, w4).astype(jnp.bfloat16)
    wk_ref[...] = _f32dot(w8, w8).astype(jnp.bfloat16)

    z1 = _f32dot(w, woh_bf).astype(jnp.bfloat16)
    z2 = _f32dot(w2, woh_bf).astype(jnp.bfloat16)
    z12 = jnp.concatenate([z1, z2], axis=1)
    z34 = _f32dot(w2, z12).astype(jnp.bfloat16)
    z14 = jnp.concatenate([z12, z34], axis=1)
    z58 = _f32dot(w4, z14).astype(jnp.bfloat16)
    z18 = jnp.concatenate([z14, z58], axis=1)
    z916 = _f32dot(w8, z18).astype(jnp.bfloat16)
    z_ref[...] = jnp.concatenate([z18, z916], axis=1)


def _local_scan_kernel(x_ref, wxp_ref, bp_ref, wbf_ref, h0_ref,
                       rall_ref, carry_ref, *, cb):
    j = pl.program_id(0)

    @pl.when(j == 0)
    def _init():
        carry_ref[...] = jnp.zeros_like(carry_ref)
        carry_ref[0:h0_ref.shape[0], :] = h0_ref[...]

    v = _f32dot(x_ref[...].reshape(cb, -1).astype(jnp.bfloat16),
                wxp_ref[...]) + bp_ref[...]
    r = _f32dot(carry_ref[...].astype(jnp.bfloat16), wbf_ref[...]) + v
    carry_ref[...] = r
    rall_ref[...] = r.reshape(rall_ref.shape).astype(jnp.bfloat16)


def _boundary_kernel(e_ref, wk_ref, sprev_ref, hfin_ref, *, n_chunks):
    wk = wk_ref[...]
    sprev_ref[0] = jnp.zeros_like(sprev_ref[0])
    s = e_ref[0]
    for c in range(1, n_chunks):
        sprev_ref[c] = s.astype(jnp.bfloat16)
        s = _f32dot(s.astype(jnp.bfloat16), wk) + e_ref[c]
    hfin_ref[...] = s


def _head_kernel(rall_ref, sprev_ref, z_ref, woh_ref, boh_ref, y_ref, *, cb):
    r = rall_ref[...].reshape(cb, -1)
    sp = sprev_ref[...].reshape(cb, -1)
    logits = (_f32dot(r, woh_ref[...]) + _f32dot(sp, z_ref[...])
              + boh_ref[...])
    m = jnp.max(logits, axis=1, keepdims=True)
    sh = logits - m
    lse = jnp.log(jnp.sum(jnp.exp(sh), axis=1, keepdims=True))
    y_ref[...] = (sh - lse).reshape(y_ref.shape)


def kernel(xs, h0, whx, bhx, whh, bhh, woh, boh):
    T, B, I = xs.shape
    H = whh.shape[0]
    O = woh.shape[1]
    K = _K
    assert T % K == 0, (T, K)
    C = T // K
    CB = C * B

    w_bf, wxp, bp, wk, z, woh_bf = pl.pallas_call(
        _prep_kernel,
        grid=(1,),
        in_specs=[
            pl.BlockSpec((I, H), lambda i: (0, 0)),
            pl.BlockSpec((1, H), lambda i: (0, 0)),
            pl.BlockSpec((H, H), lambda i: (0, 0)),
            pl.BlockSpec((1, H), lambda i: (0, 0)),
            pl.BlockSpec((H, O), lambda i: (0, 0)),
        ],
        out_specs=(
            pl.BlockSpec((H, H), lambda i: (0, 0)),
            pl.BlockSpec((I, H), lambda i: (0, 0)),
            pl.BlockSpec((1, H), lambda i: (0, 0)),
            pl.BlockSpec((H, H), lambda i: (0, 0)),
            pl.BlockSpec((H, K * O), lambda i: (0, 0)),
            pl.BlockSpec((H, O), lambda i: (0, 0)),
        ),
        out_shape=(
            jax.ShapeDtypeStruct((H, H), jnp.bfloat16),
            jax.ShapeDtypeStruct((I, H), jnp.bfloat16),
            jax.ShapeDtypeStruct((1, H), jnp.float32),
            jax.ShapeDtypeStruct((H, H), jnp.bfloat16),
            jax.ShapeDtypeStruct((H, K * O), jnp.bfloat16),
            jax.ShapeDtypeStruct((H, O), jnp.bfloat16),
        ),
        compiler_params=pltpu.CompilerParams(
            dimension_semantics=("arbitrary",)),
    )(whx, bhx, whh, bhh, woh)

    xs4 = xs.reshape(C, K, B, I)
    r_all, e_carry = pl.pallas_call(
        functools.partial(_local_scan_kernel, cb=CB),
        grid=(K,),
        in_specs=[
            pl.BlockSpec((C, 1, B, I), lambda j: (0, j, 0, 0)),
            pl.BlockSpec((I, H), lambda j: (0, 0)),
            pl.BlockSpec((1, H), lambda j: (0, 0)),
            pl.BlockSpec((H, H), lambda j: (0, 0)),
            pl.BlockSpec((B, H), lambda j: (0, 0)),
        ],
        out_specs=(
            pl.BlockSpec((1, CB, H), lambda j: (j, 0, 0)),
            pl.BlockSpec((CB, H), lambda j: (0, 0)),
        ),
        out_shape=(
            jax.ShapeDtypeStruct((K, CB, H), jnp.bfloat16),
            jax.ShapeDtypeStruct((CB, H), jnp.float32),
        ),
        compiler_params=pltpu.CompilerParams(
            dimension_semantics=("arbitrary",)),
        cost_estimate=pl.CostEstimate(
            flops=2 * T * B * H * (H + I), transcendentals=0,
            bytes_accessed=T * B * I * 4 + T * B * H * 2 + CB * H * 4),
    )(xs4, wxp, bp, w_bf, h0)

    s_prev, h_final = pl.pallas_call(
        functools.partial(_boundary_kernel, n_chunks=C),
        grid=(1,),
        in_specs=[
            pl.BlockSpec((C, B, H), lambda i: (0, 0, 0)),
            pl.BlockSpec((H, H), lambda i: (0, 0)),
        ],
        out_specs=(
            pl.BlockSpec((C, B, H), lambda i: (0, 0, 0)),
            pl.BlockSpec((B, H), lambda i: (0, 0)),
        ),
        out_shape=(
            jax.ShapeDtypeStruct((C, B, H), jnp.bfloat16),
            jax.ShapeDtypeStruct((B, H), jnp.float32),
        ),
        compiler_params=pltpu.CompilerParams(
            dimension_semantics=("arbitrary",)),
    )(e_carry.reshape(C, B, H), wk)

    y4 = pl.pallas_call(
        functools.partial(_head_kernel, cb=CB),
        grid=(K,),
        in_specs=[
            pl.BlockSpec((1, CB, H), lambda j: (j, 0, 0)),
            pl.BlockSpec((C, B, H), lambda j: (0, 0, 0)),
            pl.BlockSpec((H, O), lambda j: (0, j)),
            pl.BlockSpec((H, O), lambda j: (0, 0)),
            pl.BlockSpec((1, O), lambda j: (0, 0)),
        ],
        out_specs=pl.BlockSpec((C, 1, B, O), lambda j: (0, j, 0, 0)),
        out_shape=jax.ShapeDtypeStruct((C, K, B, O), jnp.float32),
        compiler_params=pltpu.CompilerParams(
            dimension_semantics=("arbitrary",)),
        cost_estimate=pl.CostEstimate(
            flops=4 * T * B * H * O, transcendentals=T * B * (O + 1),
            bytes_accessed=T * B * H * 2 + C * B * H * 2 + T * B * O * 4),
    )(r_all, s_prev, z, woh_bf, boh)

    return y4.reshape(T, B, O), h_final


# speedup vs baseline: 2.4281x; 1.0729x over previous
import functools

import jax
import jax.numpy as jnp
from jax.experimental import pallas as pl
from jax.experimental.pallas import tpu as pltpu


_K = 16


def _f32dot(a, b):
    return jnp.dot(a, b, preferred_element_type=jnp.float32)


def _bf16dot(a, b):
    return _f32dot(a, b).astype(jnp.bfloat16)


def _scan_kernel(x_ref, whx_ref, bhx_ref, whh_ref, bhh_ref, woh_ref, h0_ref,
                 l_ref, z_ref, sprev_ref, hfin_ref,
                 wbf, wxp, bp, wohbf, wkp, carry, *, k_steps, n_chunks, b):
    j = pl.program_id(0)
    cb = n_chunks * b

    @pl.when(j == 0)
    def _prep():
        w = whh_ref[...].astype(jnp.bfloat16)
        wbf[...] = w
        wxp[...] = _bf16dot(whx_ref[...].astype(jnp.bfloat16), w)
        bp[...] = _f32dot(bhx_ref[...].astype(jnp.bfloat16), w) + bhh_ref[...]
        woh_bf = woh_ref[...].astype(jnp.bfloat16)
        wohbf[...] = woh_bf
        w2 = _bf16dot(w, w)
        w4 = _bf16dot(w2, w2)
        z1 = _bf16dot(w, woh_bf)
        z2 = _bf16dot(w2, woh_bf)
        z12 = jnp.concatenate([z1, z2], axis=1)
        z14 = jnp.concatenate([z12, _bf16dot(w2, z12)], axis=1)
        z18 = jnp.concatenate([z14, _bf16dot(w4, z14)], axis=1)
        if k_steps == 8:
            wkp[...] = _bf16dot(w4, w4)
            z_ref[...] = z18
        else:
            w8 = _bf16dot(w4, w4)
            wkp[...] = _bf16dot(w8, w8)
            z_ref[...] = jnp.concatenate([z18, _bf16dot(w8, z18)], axis=1)
        carry[...] = jnp.zeros_like(carry)
        carry[0:b, :] = h0_ref[...]

    v = _f32dot(x_ref[...].reshape(cb, -1).astype(jnp.bfloat16),
                wxp[...]) + bp[...]
    r = _f32dot(carry[...].astype(jnp.bfloat16), wbf[...]) + v
    carry[...] = r
    l_ref[...] = _bf16dot(r.astype(jnp.bfloat16),
                          wohbf[...]).reshape(l_ref.shape)

    @pl.when(j == k_steps - 1)
    def _boundary():
        wk = wkp[...]
        sprev_ref[0] = jnp.zeros_like(sprev_ref[0])
        s = carry[0:b, :]
        for c in range(1, n_chunks):
            sprev_ref[c] = s.astype(jnp.bfloat16)
            s = _f32dot(s.astype(jnp.bfloat16), wk) + carry[c * b:(c + 1) * b, :]
        hfin_ref[...] = s


def _head_kernel(l_ref, sprev_ref, z_ref, boh_ref, y_ref, *, cb):
    sp = sprev_ref[...].reshape(cb, -1)
    logits = (l_ref[...].reshape(cb, -1).astype(jnp.float32)
              + _f32dot(sp, z_ref[...]) + boh_ref[...])
    m = jnp.max(logits, axis=1, keepdims=True)
    sh = logits - m
    lse = jnp.log(jnp.sum(jnp.exp(sh), axis=1, keepdims=True))
    y_ref[...] = (sh - lse).reshape(y_ref.shape)


def kernel(xs, h0, whx, bhx, whh, bhh, woh, boh):
    T, B, I = xs.shape
    H = whh.shape[0]
    O = woh.shape[1]
    K = _K
    assert T % K == 0, (T, K)
    C = T // K
    CB = C * B

    xs4 = xs.reshape(C, K, B, I)
    l_all, z, s_prev, h_final = pl.pallas_call(
        functools.partial(_scan_kernel, k_steps=K, n_chunks=C, b=B),
        grid=(K,),
        in_specs=[
            pl.BlockSpec((C, 1, B, I), lambda j: (0, j, 0, 0)),
            pl.BlockSpec((I, H), lambda j: (0, 0)),
            pl.BlockSpec((1, H), lambda j: (0, 0)),
            pl.BlockSpec((H, H), lambda j: (0, 0)),
            pl.BlockSpec((1, H), lambda j: (0, 0)),
            pl.BlockSpec((H, O), lambda j: (0, 0)),
            pl.BlockSpec((B, H), lambda j: (0, 0)),
        ],
        out_specs=(
            pl.BlockSpec((1, CB, O), lambda j: (j, 0, 0)),
            pl.BlockSpec((H, K * O), lambda j: (0, 0)),
            pl.BlockSpec((C, B, H), lambda j: (0, 0, 0)),
            pl.BlockSpec((B, H), lambda j: (0, 0)),
        ),
        out_shape=(
            jax.ShapeDtypeStruct((K, CB, O), jnp.bfloat16),
            jax.ShapeDtypeStruct((H, K * O), jnp.bfloat16),
            jax.ShapeDtypeStruct((C, B, H), jnp.bfloat16),
            jax.ShapeDtypeStruct((B, H), jnp.float32),
        ),
        scratch_shapes=[
            pltpu.VMEM((H, H), jnp.bfloat16),
            pltpu.VMEM((I, H), jnp.bfloat16),
            pltpu.VMEM((1, H), jnp.float32),
            pltpu.VMEM((H, O), jnp.bfloat16),
            pltpu.VMEM((H, H), jnp.bfloat16),
            pltpu.VMEM((CB, H), jnp.float32),
        ],
        compiler_params=pltpu.CompilerParams(
            dimension_semantics=("arbitrary",)),
        cost_estimate=pl.CostEstimate(
            flops=2 * T * B * H * (H + I + O) + 8 * H * H * H,
            transcendentals=0,
            bytes_accessed=(T * B * I * 4 + T * B * O * 2 + C * B * H * 2
                            + B * H * 4)),
    )(xs4, whx, bhx, whh, bhh, woh, h0)

    y4 = pl.pallas_call(
        functools.partial(_head_kernel, cb=CB),
        grid=(K,),
        in_specs=[
            pl.BlockSpec((1, CB, O), lambda j: (j, 0, 0)),
            pl.BlockSpec((C, B, H), lambda j: (0, 0, 0)),
            pl.BlockSpec((H, O), lambda j: (0, j)),
            pl.BlockSpec((1, O), lambda j: (0, 0)),
        ],
        out_specs=pl.BlockSpec((C, 1, B, O), lambda j: (0, j, 0, 0)),
        out_shape=jax.ShapeDtypeStruct((C, K, B, O), jnp.float32),
        compiler_params=pltpu.CompilerParams(
            dimension_semantics=("arbitrary",)),
        cost_estimate=pl.CostEstimate(
            flops=2 * T * B * H * O, transcendentals=T * B * (O + 1),
            bytes_accessed=T * B * O * 2 + C * B * H * 2 + T * B * O * 4),
    )(l_all, s_prev, z, boh)

    return y4.reshape(T, B, O), h_final


# speedup vs baseline: 2.6094x; 1.0746x over previous
import functools

import jax
import jax.numpy as jnp
from jax.experimental import pallas as pl
from jax.experimental.pallas import tpu as pltpu


_K = 8


def _f32dot(a, b):
    return jnp.dot(a, b, preferred_element_type=jnp.float32)


def _bf16dot(a, b):
    return _f32dot(a, b).astype(jnp.bfloat16)


def _scan_kernel(x_ref, whx_ref, bhx_ref, whh_ref, bhh_ref, woh_ref, h0_ref,
                 l_ref, z_ref, sprev_ref, hfin_ref,
                 wbf, wxp, bp, wohbf, wkp, carry, *, k_steps, n_chunks, b):
    j = pl.program_id(0)
    cb = n_chunks * b

    @pl.when(j == 0)
    def _prep():
        w = whh_ref[...].astype(jnp.bfloat16)
        wbf[...] = w
        wxp[...] = _bf16dot(whx_ref[...].astype(jnp.bfloat16), w)
        bp[...] = _f32dot(bhx_ref[...].astype(jnp.bfloat16), w) + bhh_ref[...]
        woh_bf = woh_ref[...].astype(jnp.bfloat16)
        wohbf[...] = woh_bf
        w2 = _bf16dot(w, w)
        w4 = _bf16dot(w2, w2)
        z1 = _bf16dot(w, woh_bf)
        z2 = _bf16dot(w2, woh_bf)
        z12 = jnp.concatenate([z1, z2], axis=1)
        z14 = jnp.concatenate([z12, _bf16dot(w2, z12)], axis=1)
        z18 = jnp.concatenate([z14, _bf16dot(w4, z14)], axis=1)
        if k_steps == 8:
            wkp[...] = _bf16dot(w4, w4)
            z_ref[...] = z18
        else:
            w8 = _bf16dot(w4, w4)
            wkp[...] = _bf16dot(w8, w8)
            z_ref[...] = jnp.concatenate([z18, _bf16dot(w8, z18)], axis=1)
        carry[...] = jnp.zeros_like(carry)
        carry[0:b, :] = h0_ref[...]

    v = _f32dot(x_ref[...].reshape(cb, -1).astype(jnp.bfloat16),
                wxp[...]) + bp[...]
    r = _f32dot(carry[...].astype(jnp.bfloat16), wbf[...]) + v
    carry[...] = r
    l_ref[...] = _bf16dot(r.astype(jnp.bfloat16),
                          wohbf[...]).reshape(l_ref.shape)

    @pl.when(j == k_steps - 1)
    def _boundary():
        wk = wkp[...]
        sprev_ref[0] = jnp.zeros_like(sprev_ref[0])
        s = carry[0:b, :]
        for c in range(1, n_chunks):
            sprev_ref[c] = s.astype(jnp.bfloat16)
            s = _f32dot(s.astype(jnp.bfloat16), wk) + carry[c * b:(c + 1) * b, :]
        hfin_ref[...] = s


def _head_kernel(l_ref, sprev_ref, z_ref, boh_ref, y_ref, *, cb):
    sp = sprev_ref[...].reshape(cb, -1)
    logits = (l_ref[...].reshape(cb, -1).astype(jnp.float32)
              + _f32dot(sp, z_ref[...]) + boh_ref[...])
    m = jnp.max(logits, axis=1, keepdims=True)
    sh = logits - m
    lse = jnp.log(jnp.sum(jnp.exp(sh), axis=1, keepdims=True))
    y_ref[...] = (sh - lse).reshape(y_ref.shape)


def kernel(xs, h0, whx, bhx, whh, bhh, woh, boh):
    T, B, I = xs.shape
    H = whh.shape[0]
    O = woh.shape[1]
    K = _K
    assert T % K == 0, (T, K)
    C = T // K
    CB = C * B

    xs4 = xs.reshape(C, K, B, I)
    l_all, z, s_prev, h_final = pl.pallas_call(
        functools.partial(_scan_kernel, k_steps=K, n_chunks=C, b=B),
        grid=(K,),
        in_specs=[
            pl.BlockSpec((C, 1, B, I), lambda j: (0, j, 0, 0)),
            pl.BlockSpec((I, H), lambda j: (0, 0)),
            pl.BlockSpec((1, H), lambda j: (0, 0)),
            pl.BlockSpec((H, H), lambda j: (0, 0)),
            pl.BlockSpec((1, H), lambda j: (0, 0)),
            pl.BlockSpec((H, O), lambda j: (0, 0)),
            pl.BlockSpec((B, H), lambda j: (0, 0)),
        ],
        out_specs=(
            pl.BlockSpec((1, CB, O), lambda j: (j, 0, 0)),
            pl.BlockSpec((H, K * O), lambda j: (0, 0)),
            pl.BlockSpec((C, B, H), lambda j: (0, 0, 0)),
            pl.BlockSpec((B, H), lambda j: (0, 0)),
        ),
        out_shape=(
            jax.ShapeDtypeStruct((K, CB, O), jnp.bfloat16),
            jax.ShapeDtypeStruct((H, K * O), jnp.bfloat16),
            jax.ShapeDtypeStruct((C, B, H), jnp.bfloat16),
            jax.ShapeDtypeStruct((B, H), jnp.float32),
        ),
        scratch_shapes=[
            pltpu.VMEM((H, H), jnp.bfloat16),
            pltpu.VMEM((I, H), jnp.bfloat16),
            pltpu.VMEM((1, H), jnp.float32),
            pltpu.VMEM((H, O), jnp.bfloat16),
            pltpu.VMEM((H, H), jnp.bfloat16),
            pltpu.VMEM((CB, H), jnp.float32),
        ],
        compiler_params=pltpu.CompilerParams(
            dimension_semantics=("arbitrary",)),
        cost_estimate=pl.CostEstimate(
            flops=2 * T * B * H * (H + I + O) + 8 * H * H * H,
            transcendentals=0,
            bytes_accessed=(T * B * I * 4 + T * B * O * 2 + C * B * H * 2
                            + B * H * 4)),
    )(xs4, whx, bhx, whh, bhh, woh, h0)

    y4 = pl.pallas_call(
        functools.partial(_head_kernel, cb=CB),
        grid=(K,),
        in_specs=[
            pl.BlockSpec((1, CB, O), lambda j: (j, 0, 0)),
            pl.BlockSpec((C, B, H), lambda j: (0, 0, 0)),
            pl.BlockSpec((H, O), lambda j: (0, j)),
            pl.BlockSpec((1, O), lambda j: (0, 0)),
        ],
        out_specs=pl.BlockSpec((C, 1, B, O), lambda j: (0, j, 0, 0)),
        out_shape=jax.ShapeDtypeStruct((C, K, B, O), jnp.float32),
        compiler_params=pltpu.CompilerParams(
            dimension_semantics=("arbitrary",)),
        cost_estimate=pl.CostEstimate(
            flops=2 * T * B * H * O, transcendentals=T * B * (O + 1),
            bytes_accessed=T * B * O * 2 + C * B * H * 2 + T * B * O * 4),
    )(l_all, s_prev, z, boh)

    return y4.reshape(T, B, O), h_final
